# baseline (device time: 90009 ns/iter reference)
import jax
import jax.numpy as jnp
from jax import lax
from jax.experimental import pallas as pl
from jax.experimental.pallas import tpu as pltpu

N_DEV = 8
M_BLK = 512
K_BLK = 512
BN = 256
N_TILES = 32
P = 4
KP = 4096 // P


def kernel(x, w_mat):
    m_tot, k_loc = x.shape
    k_tot, n = w_mat.shape
    xb = x.astype(jnp.bfloat16)

    def body(x_ref, w_hbm, out_ref, w_buf, gath_ref, dma_sems, cp_sems):
        def tile_copies(j, slot):
            return [
                pltpu.make_async_copy(
                    w_hbm.at[pl.ds(p * KP, KP), pl.ds(j * BN, BN)],
                    w_buf.at[slot, pl.ds(p * KP, KP), :],
                    dma_sems.at[slot, p],
                )
                for p in range(P)
            ]

        for d in range(N_DEV):
            pltpu.make_async_copy(
                x_ref.at[pl.ds(d * M_BLK, M_BLK), :],
                gath_ref.at[:, pl.ds(d * K_BLK, K_BLK)],
                cp_sems.at[d],
            ).start()
        for d in range(N_DEV):
            pltpu.make_async_copy(
                x_ref.at[pl.ds(d * M_BLK, M_BLK), :],
                gath_ref.at[:, pl.ds(d * K_BLK, K_BLK)],
                cp_sems.at[d],
            ).wait()

        for c in tile_copies(0, 0):
            c.start()
        for c in tile_copies(1, 1):
            c.start()

        def consume(j, slot):
            for c in tile_copies(j, slot):
                c.wait()
            y = jnp.dot(
                gath_ref[...], w_buf[slot].astype(jnp.bfloat16),
                preferred_element_type=jnp.float32,
            )
            out_ref[:, pl.ds(j * BN, BN)] = y * jax.nn.sigmoid(y)

            @pl.when(j + 2 < N_TILES)
            def _prefetch():
                for c in tile_copies(j + 2, slot):
                    c.start()

        def step(i, carry):
            consume(2 * i, 0)
            consume(2 * i + 1, 1)
            return carry

        lax.fori_loop(0, N_TILES // 2, step, 0)

    return pl.pallas_call(
        body,
        in_specs=[
            pl.BlockSpec(memory_space=pltpu.MemorySpace.VMEM),
            pl.BlockSpec(memory_space=pl.ANY),
        ],
        out_specs=pl.BlockSpec(memory_space=pltpu.MemorySpace.VMEM),
        out_shape=jax.ShapeDtypeStruct((M_BLK, n), jnp.float32),
        scratch_shapes=[
            pltpu.VMEM((2, k_tot, BN), jnp.float32),
            pltpu.VMEM((M_BLK, k_tot), jnp.bfloat16),
            pltpu.SemaphoreType.DMA((2, P)),
            pltpu.SemaphoreType.DMA((N_DEV,)),
        ],
    )(xb, w_mat)


# device time: 61244 ns/iter; 1.4697x vs baseline; 1.4697x over previous
import jax
import jax.numpy as jnp
from jax import lax
from jax.experimental import pallas as pl
from jax.experimental.pallas import tpu as pltpu

N_DEV = 8
M_BLK = 512
K_BLK = 512
BN = 256
N_STEPS = 16


def kernel(x, w_mat):
    m_tot, k_loc = x.shape
    k_tot, n = w_mat.shape
    xb = x.astype(jnp.bfloat16)

    def body(x_ref, wa_ref, wb_ref, out_ref, gath_ref, cp_sems):
        tn = pl.program_id(0)

        @pl.when(tn == 0)
        def _build_gather():
            for d in range(N_DEV):
                pltpu.make_async_copy(
                    x_ref.at[pl.ds(d * M_BLK, M_BLK), :],
                    gath_ref.at[:, pl.ds(d * K_BLK, K_BLK)],
                    cp_sems.at[d],
                ).start()
            for d in range(N_DEV):
                pltpu.make_async_copy(
                    x_ref.at[pl.ds(d * M_BLK, M_BLK), :],
                    gath_ref.at[:, pl.ds(d * K_BLK, K_BLK)],
                    cp_sems.at[d],
                ).wait()

        ya = jnp.dot(
            gath_ref[...], wa_ref[...].astype(jnp.bfloat16),
            preferred_element_type=jnp.float32,
        )
        out_ref[:, pl.ds(0, BN)] = ya * jax.nn.sigmoid(ya)
        yb = jnp.dot(
            gath_ref[...], wb_ref[...].astype(jnp.bfloat16),
            preferred_element_type=jnp.float32,
        )
        out_ref[:, pl.ds(BN, BN)] = yb * jax.nn.sigmoid(yb)

    return pl.pallas_call(
        body,
        grid=(N_STEPS,),
        in_specs=[
            pl.BlockSpec((m_tot, K_BLK), lambda tn: (0, 0)),
            pl.BlockSpec((k_tot, BN), lambda tn: (0, 2 * tn)),
            pl.BlockSpec((k_tot, BN), lambda tn: (0, 2 * tn + 1)),
        ],
        out_specs=pl.BlockSpec((M_BLK, 2 * BN), lambda tn: (0, tn)),
        out_shape=jax.ShapeDtypeStruct((M_BLK, n), jnp.float32),
        scratch_shapes=[
            pltpu.VMEM((M_BLK, k_tot), jnp.bfloat16),
            pltpu.SemaphoreType.DMA((N_DEV,)),
        ],
        compiler_params=pltpu.CompilerParams(
            dimension_semantics=("arbitrary",),
        ),
    )(xb, w_mat, w_mat)
